# baseline (speedup 1.0000x reference)
import jax
import jax.numpy as jnp
from jax import lax
from jax.experimental import pallas as pl
from jax.experimental.pallas import tpu as pltpu
from jax.experimental.pallas import tpu_sc as plsc

SHIFT = 500000
B = 16384
D = 64
L = 16
NC = 2
NS = 16
NW = NC * NS
BPW = B // NW
NCHUNK = 4
CHUNK = BPW // NCHUNK


def _rsqrt(x):
    i = lax.bitcast_convert_type(x, jnp.int32)
    i = jnp.int32(0x5F3759DF) - lax.shift_right_logical(i, 1)
    y = lax.bitcast_convert_type(i, jnp.float32)
    y = y * (1.5 - 0.5 * x * y * y)
    y = y * (1.5 - 0.5 * x * y * y)
    return y


def _body(offset_hbm, table_hbm, out_hbm, i0, i1, i2, i3, rows_v, sem):
    idx_refs = [i0, i1, i2, i3]
    wid = lax.axis_index("s") * NC + lax.axis_index("c")
    base = wid * BPW

    for j in range(NCHUNK):
        pltpu.sync_copy(offset_hbm.at[pl.ds(base + j * CHUNK, CHUNK)],
                        idx_refs[j])
    for j in range(NCHUNK):
        for i in range(CHUNK // L):
            sl = pl.ds(i * L, L)
            idx_refs[j][sl] = idx_refs[j][sl] + SHIFT

    descs = [
        pltpu.async_copy(table_hbm.at[idx_refs[j]],
                         rows_v.at[pl.ds(j * CHUNK, CHUNK)], sem)
        for j in range(NCHUNK)
    ]
    for d in descs:
        d.wait()

    @plsc.parallel_loop(0, BPW, unroll=2)
    def _row(r):
        c = [rows_v[r, pl.ds(k * L, L)] for k in range(D // L)]
        acc = c[0] * c[0]
        for k in range(1, D // L):
            acc = acc + c[k] * c[k]
        s = jnp.sum(acc)
        sv = jnp.broadcast_to(s, (L,))
        scale = jnp.minimum(1.0, 2.0 * _rsqrt(sv))
        for k in range(D // L):
            rows_v[r, pl.ds(k * L, L)] = c[k] * scale

    pltpu.sync_copy(rows_v, out_hbm.at[pl.ds(base, BPW)])


@jax.jit
def kernel(offset, table):
    mesh = plsc.VectorSubcoreMesh(core_axis_name="c", subcore_axis_name="s",
                                  num_cores=NC, num_subcores=NS)
    run = pl.kernel(
        _body,
        out_type=jax.ShapeDtypeStruct((B, D), jnp.float32),
        mesh=mesh,
        scratch_types=[pltpu.VMEM((CHUNK,), jnp.int32)] * NCHUNK + [
            pltpu.VMEM((BPW, D), jnp.float32),
            pltpu.SemaphoreType.DMA,
        ],
        compiler_params=pltpu.CompilerParams(needs_layout_passes=False,
                                             use_tc_tiling_on_sc=False),
    )
    return run(offset, table)


# speedup vs baseline: 2.1715x; 2.1715x over previous
import jax
import jax.numpy as jnp
from jax import lax
from jax.experimental import pallas as pl
from jax.experimental.pallas import tpu as pltpu
from jax.experimental.pallas import tpu_sc as plsc

SHIFT = 500000
B = 16384
D = 64
V = 1000000
L = 16
NC = 2
NS = 16
NW = NC * NS
BPW = B // NW
G = 32
NCH = BPW // G
TR = 8


def _rsqrt(x):
    i = lax.bitcast_convert_type(x, jnp.int32)
    i = jnp.int32(0x5F3759DF) - lax.shift_right_logical(i, 1)
    y = lax.bitcast_convert_type(i, jnp.float32)
    y = y * (1.5 - 0.5 * x * y * y)
    y = y * (1.5 - 0.5 * x * y * y)
    return y


def _body(offset_hbm, table_hbm, out_hbm,
          off_v, tidx_v, rmod_v, gbuf, stage, sem):
    wid = lax.axis_index("s") * NC + lax.axis_index("c")
    base = wid * BPW

    pltpu.sync_copy(offset_hbm.at[pl.ds(base, BPW)], off_v)
    for i in range(BPW // L):
        v = off_v[pl.ds(i * L, L)] + SHIFT
        tidx_v[pl.ds(i * L, L)] = lax.shift_right_logical(v, 3)
        rmod_v[pl.ds(i * L, L)] = v & 7

    @pl.loop(0, NCH)
    def _chunk(c):
        descs = []
        for g in range(G // L):
            tv = tidx_v[pl.ds(c * G + g * L, L)]
            for j in range(L):
                descs.append(pltpu.async_copy(
                    table_hbm.at[tv[j]], gbuf.at[g * L + j], sem))
        for d in descs:
            d.wait()

        for g in range(G // L):
            rv = rmod_v[pl.ds(c * G + g * L, L)]
            for j in range(L):
                jj = g * L + j
                row = c * G + jj
                r = rv[j]
                ck = [gbuf[jj, r, pl.ds(k * L, L)] for k in range(D // L)]
                acc = ck[0] * ck[0]
                for k in range(1, D // L):
                    acc = acc + ck[k] * ck[k]
                s = jnp.sum(acc)
                sv = jnp.broadcast_to(s, (L,))
                scale = jnp.minimum(1.0, 2.0 * _rsqrt(sv))
                for k in range(D // L):
                    stage[lax.shift_right_logical(row, 3), row & 7,
                          pl.ds(k * L, L)] = ck[k] * scale

    pltpu.sync_copy(stage, out_hbm.at[pl.ds(wid * (BPW // TR), BPW // TR)])


@jax.jit
def kernel(offset, table):
    table3 = table.reshape(V // TR, TR, D)
    mesh = plsc.VectorSubcoreMesh(core_axis_name="c", subcore_axis_name="s",
                                  num_cores=NC, num_subcores=NS)
    run = pl.kernel(
        _body,
        out_type=jax.ShapeDtypeStruct((B // TR, TR, D), jnp.float32),
        mesh=mesh,
        scratch_types=[
            pltpu.VMEM((BPW,), jnp.int32),
            pltpu.VMEM((BPW,), jnp.int32),
            pltpu.VMEM((BPW,), jnp.int32),
            pltpu.VMEM((G, TR, D), jnp.float32),
            pltpu.VMEM((BPW // TR, TR, D), jnp.float32),
            pltpu.SemaphoreType.DMA,
        ],
        compiler_params=pltpu.CompilerParams(needs_layout_passes=False,
                                             use_tc_tiling_on_sc=True),
    )
    out3 = run(offset, table3)
    return out3.reshape(B, D)
